# baseline (speedup 1.0000x reference)
import functools

import jax
import jax.numpy as jnp
from jax import lax
from jax.experimental import pallas as pl
from jax.experimental.pallas import tpu as pltpu
from jax.experimental.pallas import tpu_sc as plsc

_B = 16384
_F = 26
_V = 100000
_D = 16

_NC = 2
_NS = 16
_NW = _NC * _NS
_N = _B * _F
_RPW = _N // _NW
_CH = 1024
_NCHUNK = _RPW // _CH


def _gather_body(idx_hbm, tab_hbm, out_hbm, idx_v, rows_v, sem):
    wid = lax.axis_index("s") * _NC + lax.axis_index("c")
    base = wid * _RPW

    def step(g, carry):
        off = base + g * _CH
        pltpu.sync_copy(idx_hbm.at[pl.ds(off, _CH)], idx_v)
        pltpu.async_copy(tab_hbm.at[idx_v], rows_v, sem).wait()
        pltpu.sync_copy(rows_v, out_hbm.at[pl.ds(off, _CH)])
        return carry

    lax.fori_loop(0, _NCHUNK, step, 0)


_gather = functools.partial(
    pl.kernel,
    mesh=plsc.VectorSubcoreMesh(core_axis_name="c", subcore_axis_name="s"),
    out_type=jax.ShapeDtypeStruct((_N, _D), jnp.float32),
    scratch_types=[
        pltpu.VMEM((_CH,), jnp.int32),
        pltpu.VMEM((_CH, _D), jnp.float32),
        pltpu.SemaphoreType.DMA,
    ],
    compiler_params=pltpu.CompilerParams(use_tc_tiling_on_sc=False),
)(_gather_body)


@jax.jit
def kernel(indices, tables):
    flat_idx = (indices + (jnp.arange(_F, dtype=jnp.int32) * _V)[None, :]).reshape(_N)
    flat_tab = tables.reshape(_F * _V, _D)
    out = _gather(flat_idx, flat_tab)
    return out.reshape(_B, _F * _D)


# speedup vs baseline: 1.0154x; 1.0154x over previous
import functools

import jax
import jax.numpy as jnp
from jax import lax
from jax.experimental import pallas as pl
from jax.experimental.pallas import tpu as pltpu
from jax.experimental.pallas import tpu_sc as plsc

_B = 16384
_F = 26
_V = 100000
_D = 16

_NC = 2
_NS = 16
_NW = _NC * _NS
_N = _B * _F
_RPW = _N // _NW
_CH = 1024
_NCHUNK = _RPW // _CH
_NBUF = 4


def _gather_body(idx_hbm, tab_hbm, out_hbm, idx_v, rows_v, gsem):
    wid = lax.axis_index("s") * _NC + lax.axis_index("c")
    base = wid * _RPW

    for b in range(_NBUF):
        pltpu.sync_copy(idx_hbm.at[pl.ds(base + b * _CH, _CH)], idx_v.at[b])
        pltpu.async_copy(tab_hbm.at[idx_v.at[b]], rows_v.at[b], gsem)

    for g in range(_NCHUNK):
        b = g % _NBUF
        pltpu.make_async_copy(tab_hbm.at[idx_v.at[b]], rows_v.at[b], gsem).wait()
        pltpu.sync_copy(rows_v.at[b], out_hbm.at[pl.ds(base + g * _CH, _CH)])
        n = g + _NBUF
        if n < _NCHUNK:
            pltpu.sync_copy(idx_hbm.at[pl.ds(base + n * _CH, _CH)], idx_v.at[b])
            pltpu.async_copy(tab_hbm.at[idx_v.at[b]], rows_v.at[b], gsem)


_gather = functools.partial(
    pl.kernel,
    mesh=plsc.VectorSubcoreMesh(core_axis_name="c", subcore_axis_name="s"),
    out_type=jax.ShapeDtypeStruct((_N, _D), jnp.float32),
    scratch_types=[
        pltpu.VMEM((_NBUF, _CH), jnp.int32),
        pltpu.VMEM((_NBUF, _CH, _D), jnp.float32),
        pltpu.SemaphoreType.DMA,
    ],
    compiler_params=pltpu.CompilerParams(use_tc_tiling_on_sc=False),
)(_gather_body)


@jax.jit
def kernel(indices, tables):
    flat_idx = (indices + (jnp.arange(_F, dtype=jnp.int32) * _V)[None, :]).reshape(_N)
    flat_tab = tables.reshape(_F * _V, _D)
    out = _gather(flat_idx, flat_tab)
    return out.reshape(_B, _F * _D)


# speedup vs baseline: 5.6609x; 5.5751x over previous
import functools

import jax
import jax.numpy as jnp
from jax import lax
from jax.experimental import pallas as pl
from jax.experimental.pallas import tpu as pltpu
from jax.experimental.pallas import tpu_sc as plsc

_B = 16384
_F = 26
_V = 100000
_D = 16

_NC = 2
_NS = 16
_NW = _NC * _NS
_NT = _F * _D
_TPW = _NT // _NW
_HB = _B // 2


def _lookup_body(idx_hbm, tab_hbm, out_hbm, tabv, idxv, outv):
    wid = lax.axis_index("s") * _NC + lax.axis_index("c")

    def task(k, carry):
        t = wid + k * _NW
        f = t // _D
        d = t % _D
        pltpu.sync_copy(tab_hbm.at[f, d, :], tabv)
        for h in range(2):
            b0 = h * _HB
            pltpu.sync_copy(idx_hbm.at[f, pl.ds(b0, _HB)], idxv)

            def gath(i, c):
                vi = idxv[pl.ds(i * 16, 16)]
                outv[pl.ds(i * 16, 16)] = plsc.load_gather(tabv, [vi])
                return c

            lax.fori_loop(0, _HB // 16, gath, 0)
            pltpu.sync_copy(outv, out_hbm.at[t, pl.ds(b0, _HB)])
        return carry

    lax.fori_loop(0, _TPW, task, 0)


_lookup = functools.partial(
    pl.kernel,
    mesh=plsc.VectorSubcoreMesh(core_axis_name="c", subcore_axis_name="s"),
    out_type=jax.ShapeDtypeStruct((_NT, _B), jnp.float32),
    scratch_types=[
        pltpu.VMEM((_V,), jnp.float32),
        pltpu.VMEM((_HB,), jnp.int32),
        pltpu.VMEM((_HB,), jnp.float32),
    ],
    compiler_params=pltpu.CompilerParams(
        use_tc_tiling_on_sc=True, needs_layout_passes=False
    ),
)(_lookup_body)


@jax.jit
def kernel(indices, tables):
    idx_t = indices.T
    tab_t = tables.transpose(0, 2, 1)
    out_t = _lookup(idx_t, tab_t)
    return out_t.T.reshape(_B, _F * _D)


# speedup vs baseline: 9.3548x; 1.6525x over previous
import functools

import jax
import jax.numpy as jnp
from jax import lax
from jax.experimental import pallas as pl
from jax.experimental.pallas import tpu as pltpu
from jax.experimental.pallas import tpu_sc as plsc

_B = 16384
_F = 26
_V = 100000
_D = 16

_NC = 2
_NS = 16
_NW = _NC * _NS
_NT = _F * _D
_TPW = _NT // _NW
_PH = 4096
_NPH = _B // _PH


def _lookup_body(
    idx_hbm, tab_hbm, out_hbm, tabv, idxv0, idxv1, outv0, outv1, isem, wsem
):
    wid = lax.axis_index("s") * _NC + lax.axis_index("c")
    idxv = (idxv0, idxv1)
    outv = (outv0, outv1)

    def task(k, carry):
        t = wid + k * _NW
        f = t // _D
        d = t % _D
        pltpu.async_copy(idx_hbm.at[f, pl.ds(0, _PH)], idxv[0], isem)
        pltpu.async_copy(idx_hbm.at[f, pl.ds(_PH, _PH)], idxv[1], isem)
        pltpu.sync_copy(tab_hbm.at[f, d, :], tabv)
        for p in range(_NPH):
            b = p % 2
            iv = idxv[b]
            ov = outv[b]
            pltpu.make_async_copy(
                idx_hbm.at[f, pl.ds(p * _PH, _PH)], iv, isem
            ).wait()
            if p >= 2:
                pltpu.make_async_copy(
                    ov, out_hbm.at[t, pl.ds((p - 2) * _PH, _PH)], wsem
                ).wait()

            @plsc.parallel_loop(0, _PH, 16, unroll=8)
            def gath(i):
                ov[pl.ds(i, 16)] = plsc.load_gather(tabv, [iv[pl.ds(i, 16)]])

            if p + 2 < _NPH:
                pltpu.async_copy(
                    idx_hbm.at[f, pl.ds((p + 2) * _PH, _PH)], iv, isem
                )
            pltpu.async_copy(ov, out_hbm.at[t, pl.ds(p * _PH, _PH)], wsem)
        for p in range(_NPH - 2, _NPH):
            pltpu.make_async_copy(
                outv[p % 2], out_hbm.at[t, pl.ds(p * _PH, _PH)], wsem
            ).wait()
        return carry

    lax.fori_loop(0, _TPW, task, 0)


_lookup = functools.partial(
    pl.kernel,
    mesh=plsc.VectorSubcoreMesh(core_axis_name="c", subcore_axis_name="s"),
    out_type=jax.ShapeDtypeStruct((_NT, _B), jnp.float32),
    scratch_types=[
        pltpu.VMEM((_V,), jnp.float32),
        pltpu.VMEM((_PH,), jnp.int32),
        pltpu.VMEM((_PH,), jnp.int32),
        pltpu.VMEM((_PH,), jnp.float32),
        pltpu.VMEM((_PH,), jnp.float32),
        pltpu.SemaphoreType.DMA,
        pltpu.SemaphoreType.DMA,
    ],
    compiler_params=pltpu.CompilerParams(
        use_tc_tiling_on_sc=True, needs_layout_passes=False
    ),
)(_lookup_body)


@jax.jit
def kernel(indices, tables):
    idx_t = indices.T
    tab_t = tables.transpose(0, 2, 1)
    out_t = _lookup(idx_t, tab_t)
    return out_t.T.reshape(_B, _F * _D)


# speedup vs baseline: 10.6081x; 1.1340x over previous
import functools

import jax
import jax.numpy as jnp
from jax import lax
from jax.experimental import pallas as pl
from jax.experimental.pallas import tpu as pltpu
from jax.experimental.pallas import tpu_sc as plsc

_B = 16384
_F = 26
_V = 100000
_D = 16

_NC = 2
_NS = 16
_NW = _NC * _NS
_NT = _F * _D
_TPW = _NT // _NW
_PH = 4096
_NPH = _B // _PH


def _lookup_body(idx_hbm, tab_hbm, out_hbm, tabv, idxv, outv0, outv1, wsem):
    wid = lax.axis_index("s") * _NC + lax.axis_index("c")
    outv = (outv0, outv1)

    def task(k, carry):
        t = wid * _TPW + k
        f = t // _D
        d = t % _D
        cp = pltpu.make_async_copy(tab_hbm.at[f, d, :], tabv, wsem)
        cp.start()

        @pl.when((k == 0) | (d == 0))
        def _():
            pltpu.sync_copy(idx_hbm.at[f, :], idxv)

        cp.wait()
        for p in range(_NPH):
            b = p % 2
            ov = outv[b]
            if p >= 2:
                pltpu.make_async_copy(
                    ov, out_hbm.at[t, pl.ds((p - 2) * _PH, _PH)], wsem
                ).wait()
            p0 = p * _PH

            @plsc.parallel_loop(0, _PH, 16, unroll=8)
            def gath(i):
                ov[pl.ds(i, 16)] = plsc.load_gather(
                    tabv, [idxv[pl.ds(p0 + i, 16)]]
                )

            pltpu.async_copy(ov, out_hbm.at[t, pl.ds(p0, _PH)], wsem)
        for p in range(_NPH - 2, _NPH):
            pltpu.make_async_copy(
                outv[p % 2], out_hbm.at[t, pl.ds(p * _PH, _PH)], wsem
            ).wait()
        return carry

    lax.fori_loop(0, _TPW, task, 0)


_lookup = functools.partial(
    pl.kernel,
    mesh=plsc.VectorSubcoreMesh(core_axis_name="c", subcore_axis_name="s"),
    out_type=jax.ShapeDtypeStruct((_NT, _B), jnp.float32),
    scratch_types=[
        pltpu.VMEM((_V,), jnp.float32),
        pltpu.VMEM((_B,), jnp.int32),
        pltpu.VMEM((_PH,), jnp.float32),
        pltpu.VMEM((_PH,), jnp.float32),
        pltpu.SemaphoreType.DMA,
    ],
    compiler_params=pltpu.CompilerParams(
        use_tc_tiling_on_sc=True, needs_layout_passes=False
    ),
)(_lookup_body)


@jax.jit
def kernel(indices, tables):
    idx_t = indices.T
    tab_t = tables.transpose(0, 2, 1)
    out_t = _lookup(idx_t, tab_t)
    return out_t.T.reshape(_B, _F * _D)


# speedup vs baseline: 10.7712x; 1.0154x over previous
import functools

import jax
import jax.numpy as jnp
from jax import lax
from jax.experimental import pallas as pl
from jax.experimental.pallas import tpu as pltpu
from jax.experimental.pallas import tpu_sc as plsc

_B = 16384
_F = 26
_V = 100000
_D = 16

_NC = 2
_NS = 16
_NW = _NC * _NS
_NT = _F * _D
_TPW = _NT // _NW
_PH = 4096
_NPH = _B // _PH


def _lookup_body(idx_hbm, tab_hbm, out_hbm, tabv, idxv, outv0, outv1, wsem, tsem):
    wid = lax.axis_index("s") * _NC + lax.axis_index("c")
    outv = (outv0, outv1)

    def task(k, carry):
        t = wid * _TPW + k
        f = t // _D
        d = t % _D
        cp = pltpu.make_async_copy(tab_hbm.at[f, d, :], tabv, tsem)
        cp.start()

        @pl.when((k == 0) | (d == 0))
        def _():
            pltpu.sync_copy(idx_hbm.at[f, :], idxv)

        cp.wait()
        for p in range(_NPH):
            b = p % 2
            ov = outv[b]
            p0 = p * _PH

            @pl.when((k > 0) | (p >= 2))
            def _():
                pltpu.make_async_copy(
                    ov, out_hbm.at[t, pl.ds(p0, _PH)], wsem
                ).wait()

            @plsc.parallel_loop(0, _PH, 16, unroll=8)
            def gath(i):
                ov[pl.ds(i, 16)] = plsc.load_gather(
                    tabv, [idxv[pl.ds(p0 + i, 16)]]
                )

            pltpu.async_copy(ov, out_hbm.at[t, pl.ds(p0, _PH)], wsem)
        return carry

    lax.fori_loop(0, _TPW, task, 0)
    t_last = wid * _TPW + (_TPW - 1)
    for p in range(_NPH - 2, _NPH):
        pltpu.make_async_copy(
            outv[p % 2], out_hbm.at[t_last, pl.ds(p * _PH, _PH)], wsem
        ).wait()


_lookup = functools.partial(
    pl.kernel,
    mesh=plsc.VectorSubcoreMesh(core_axis_name="c", subcore_axis_name="s"),
    out_type=jax.ShapeDtypeStruct((_NT, _B), jnp.float32),
    scratch_types=[
        pltpu.VMEM((_V,), jnp.float32),
        pltpu.VMEM((_B,), jnp.int32),
        pltpu.VMEM((_PH,), jnp.float32),
        pltpu.VMEM((_PH,), jnp.float32),
        pltpu.SemaphoreType.DMA,
        pltpu.SemaphoreType.DMA,
    ],
    compiler_params=pltpu.CompilerParams(
        use_tc_tiling_on_sc=True, needs_layout_passes=False
    ),
)(_lookup_body)


@jax.jit
def kernel(indices, tables):
    idx_t = indices.T
    tab_t = tables.transpose(0, 2, 1)
    out_t = _lookup(idx_t, tab_t)
    return out_t.T.reshape(_B, _F * _D)
